# baseline (speedup 1.0000x reference)
import functools

import jax
import jax.numpy as jnp
import numpy as np
from jax.experimental import pallas as pl
from jax.experimental.pallas import tpu as pltpu

_N = 4096
_D = 128
_CUTOFF = 5.0
_NFREQ = 20
_RB = 128
_CW = 192
_NRB = _N // _RB
_NP = _N + _CW
_KW = (_NFREQ + 1) * _CW


def _phi_kernel(z_ref, emb_ref, w1_ref, b1_ref, w2_ref, b2_ref, phi_ref):
    zb = z_ref[...]
    lane = jax.lax.broadcasted_iota(jnp.int32, (zb.shape[0], 128), 1)
    onehot = (zb == lane).astype(jnp.float32)
    s = jnp.dot(onehot, emb_ref[...], preferred_element_type=jnp.float32)
    h = jnp.dot(s, w1_ref[...], preferred_element_type=jnp.float32) + b1_ref[...]
    h = h * jax.nn.sigmoid(h)
    phi_ref[...] = (jnp.dot(h, w2_ref[...], preferred_element_type=jnp.float32)
                    + b2_ref[...]).astype(jnp.bfloat16)


def _pair_kernel(meta_ref, pos_row_ref, batch_row_ref, posc_ref, batchc_ref,
                 phi1_ref, phi3_ref, w1_ref, w3_ref, b1_ref, b3_ref,
                 ds_ref, dv_ref, ls_a, lv_a, rs_a, rv_a):
    r = pl.program_id(0)
    start_col = meta_ref[0, r]
    num_chunks = meta_ref[1, r]

    pos_row = pos_row_ref[...]
    batch_row = batch_row_ref[...]
    sq_i = jnp.sum(pos_row * pos_row, axis=1, keepdims=True)
    row_ids = r * _RB + jax.lax.broadcasted_iota(jnp.int32, (_RB, _CW), 0)

    b1 = b1_ref[...]
    b3 = b3_ref[...]
    bf = jnp.bfloat16

    def build(w, ls_ref, lv_ref, rs_ref, rv_ref):
        w = pl.multiple_of(w, 8)
        posc = posc_ref[pl.ds(w, _CW), :].T
        batchc = batchc_ref[pl.ds(w, _CW), :].reshape(1, _CW)
        phi1 = phi1_ref[pl.ds(w, _CW), :]
        phi3 = phi3_ref[pl.ds(w, _CW), :]

        sq_j = jnp.sum(posc * posc, axis=0, keepdims=True)
        cross = jnp.dot(pos_row, posc, preferred_element_type=jnp.float32)
        d2m = jnp.maximum(sq_i + sq_j - 2.0 * cross, 0.0)
        dm = jnp.sqrt(d2m)

        col_ids = w + jax.lax.broadcasted_iota(jnp.int32, (_RB, _CW), 1)
        m = ((dm <= _CUTOFF)
             & (batch_row == batchc)
             & (row_ids != col_ids))
        mf = m.astype(jnp.float32)

        rel0 = pos_row[:, 0:1] - posc[0:1, :]
        rel1 = pos_row[:, 1:2] - posc[1:2, :]
        rel2 = pos_row[:, 2:3] - posc[2:3, :]
        d2g = rel0 * rel0 + rel1 * rel1 + rel2 * rel2
        dg = jnp.sqrt(d2g)

        d_safe = jnp.where(m, dg, 1.0)
        invd_b = (mf / d_safe).astype(bf)
        relm0_b = (rel0 * mf).astype(bf)
        relm1_b = (rel1 * mf).astype(bf)
        relm2_b = (rel2 * mf).astype(bf)
        dg_b = dg.astype(bf)

        x = dg * (np.pi / _CUTOFF)
        s_cur = jnp.sin(x)
        cos2 = 2.0 * jnp.cos(x)
        s_prev = jnp.zeros_like(s_cur)

        for k in range(_NFREQ):
            ks = slice(k * _CW, (k + 1) * _CW)
            s_b = s_cur.astype(bf)
            ls_ref[:, ks] = s_b * invd_b
            lv_ref[0 * _RB:1 * _RB, ks] = s_b * relm0_b
            lv_ref[1 * _RB:2 * _RB, ks] = s_b * relm1_b
            lv_ref[2 * _RB:3 * _RB, ks] = s_b * relm2_b
            rs_ref[ks, :] = phi1 * w1_ref[k:k + 1, :]
            rv_ref[ks, :] = phi3 * w3_ref[k:k + 1, :]
            s_prev, s_cur = s_cur, cos2 * s_cur - s_prev
        kb = slice(_NFREQ * _CW, (_NFREQ + 1) * _CW)
        ls_ref[:, kb] = mf.astype(bf)
        lv_ref[0 * _RB:1 * _RB, kb] = dg_b * relm0_b
        lv_ref[1 * _RB:2 * _RB, kb] = dg_b * relm1_b
        lv_ref[2 * _RB:3 * _RB, kb] = dg_b * relm2_b
        rs_ref[kb, :] = phi1 * b1
        rv_ref[kb, :] = phi3 * b3

    def chunk_body(t, carry):
        acc_s, acc_v = carry
        build(start_col + t * _CW, ls_a, lv_a, rs_a, rv_a)
        acc_s = acc_s + jnp.dot(ls_a[...], rs_a[...],
                                preferred_element_type=jnp.float32)
        acc_v = acc_v + jnp.dot(lv_a[...], rv_a[...],
                                preferred_element_type=jnp.float32)
        return acc_s, acc_v

    acc_s, acc_v = jax.lax.fori_loop(
        0, num_chunks, chunk_body,
        (jnp.zeros((_RB, _D), jnp.float32),
         jnp.zeros((3 * _RB, _D), jnp.float32)))

    ds_ref[...] = acc_s
    dv_ref[0] = acc_v[0 * _RB:1 * _RB, :]
    dv_ref[1] = acc_v[1 * _RB:2 * _RB, :]
    dv_ref[2] = acc_v[2 * _RB:3 * _RB, :]


@functools.partial(jax.jit, static_argnums=())
def kernel(z, pos, batch, emb_table, W_phi1, b_phi1, W_phi2, b_phi2, W_rbf, b_rbf):
    z = z.astype(jnp.int32)
    batch = batch.astype(jnp.int32)

    emb_p = jnp.zeros((128, _D), jnp.float32).at[:emb_table.shape[0]].set(emb_table)
    w2r = jnp.concatenate([W_phi2[:, :_D], W_phi2[:, 2 * _D:]], axis=1)
    b2r = jnp.concatenate([b_phi2[:_D], b_phi2[2 * _D:]]).reshape(1, 2 * _D)
    phi = pl.pallas_call(
        _phi_kernel,
        grid=(_N // 256,),
        in_specs=[
            pl.BlockSpec((256, 1), lambda i: (i, 0)),
            pl.BlockSpec((128, _D), lambda i: (0, 0)),
            pl.BlockSpec((_D, _D), lambda i: (0, 0)),
            pl.BlockSpec((1, _D), lambda i: (0, 0)),
            pl.BlockSpec((_D, 2 * _D), lambda i: (0, 0)),
            pl.BlockSpec((1, 2 * _D), lambda i: (0, 0)),
        ],
        out_specs=pl.BlockSpec((256, 2 * _D), lambda i: (i, 0)),
        out_shape=jax.ShapeDtypeStruct((_N, 2 * _D), jnp.bfloat16),
    )(z.reshape(_N, 1), emb_p, W_phi1, b_phi1.reshape(1, _D), w2r, b2r)

    phi1 = jnp.zeros((_NP, _D), jnp.bfloat16).at[:_N].set(phi[:, :_D])
    phi3 = jnp.zeros((_NP, _D), jnp.bfloat16).at[:_N].set(phi[:, _D:])
    posc = jnp.zeros((_NP, 3), jnp.float32).at[:_N].set(pos)
    batchc = jnp.full((_NP, 1), -1, jnp.int32).at[:_N, 0].set(batch)

    b_first = batch[::_RB]
    b_last = batch[_RB - 1::_RB]
    c_lo = jnp.searchsorted(batch, b_first, side="left").astype(jnp.int32)
    c_hi = jnp.searchsorted(batch, b_last, side="right").astype(jnp.int32)
    start_col = (c_lo // 8) * 8
    num_chunks = (c_hi - start_col + _CW - 1) // _CW
    meta = jnp.stack([start_col, num_chunks]).astype(jnp.int32)

    w1 = W_rbf[:, :_D].astype(jnp.bfloat16)
    w3 = W_rbf[:, 2 * _D:].astype(jnp.bfloat16)
    b1 = b_rbf[:_D].reshape(1, _D).astype(jnp.bfloat16)
    b3 = b_rbf[2 * _D:].reshape(1, _D).astype(jnp.bfloat16)

    grid_spec = pltpu.PrefetchScalarGridSpec(
        num_scalar_prefetch=1,
        grid=(_NRB,),
        in_specs=[
            pl.BlockSpec((_RB, 3), lambda r, *_: (r, 0)),
            pl.BlockSpec((_RB, 1), lambda r, *_: (r, 0)),
            pl.BlockSpec((_NP, 3), lambda r, *_: (0, 0)),
            pl.BlockSpec((_NP, 1), lambda r, *_: (0, 0)),
            pl.BlockSpec((_NP, _D), lambda r, *_: (0, 0)),
            pl.BlockSpec((_NP, _D), lambda r, *_: (0, 0)),
            pl.BlockSpec((_NFREQ, _D), lambda r, *_: (0, 0)),
            pl.BlockSpec((_NFREQ, _D), lambda r, *_: (0, 0)),
            pl.BlockSpec((1, _D), lambda r, *_: (0, 0)),
            pl.BlockSpec((1, _D), lambda r, *_: (0, 0)),
        ],
        out_specs=[
            pl.BlockSpec((_RB, _D), lambda r, *_: (r, 0)),
            pl.BlockSpec((3, _RB, _D), lambda r, *_: (0, r, 0)),
        ],
        scratch_shapes=[
            pltpu.VMEM((_RB, _KW), jnp.bfloat16),
            pltpu.VMEM((3 * _RB, _KW), jnp.bfloat16),
            pltpu.VMEM((_KW, _D), jnp.bfloat16),
            pltpu.VMEM((_KW, _D), jnp.bfloat16),
        ],
    )
    delta_s, delta_v = pl.pallas_call(
        _pair_kernel,
        grid_spec=grid_spec,
        out_shape=[
            jax.ShapeDtypeStruct((_N, _D), jnp.float32),
            jax.ShapeDtypeStruct((3, _N, _D), jnp.float32),
        ],
    )(meta, pos, batch.reshape(_N, 1), posc, batchc, phi1, phi3, w1, w3, b1, b3)

    return delta_s, delta_v


# speedup vs baseline: 1.1208x; 1.1208x over previous
import functools

import jax
import jax.numpy as jnp
import numpy as np
from jax import lax
from jax.experimental import pallas as pl
from jax.experimental.pallas import tpu as pltpu
from jax.experimental.pallas import tpu_sc as plsc

_N = 4096
_D = 128
_CUTOFF = 5.0
_NFREQ = 20
_RB = 128
_CW = 256
_NRB = _N // _RB
_NP = _N + _CW
_KW = (_NFREQ + 1) * _CW


def _make_emb_gather():
    info = plsc.get_sparse_core_info()
    nw = info.num_cores * info.num_subcores
    b_per_w = _N // nw
    mesh = plsc.VectorSubcoreMesh(core_axis_name="c", subcore_axis_name="s")

    @functools.partial(
        pl.kernel, mesh=mesh,
        out_type=jax.ShapeDtypeStruct((_N, _D), jnp.float32),
        scratch_types=[
            pltpu.VMEM((b_per_w,), jnp.int32),
            pltpu.VMEM((b_per_w, _D), jnp.float32),
            pltpu.SemaphoreType.DMA,
        ],
    )
    def gather_kernel(table_hbm, idx_hbm, out_hbm, idx_v, rows_v, sem):
        wid = lax.axis_index("s") * info.num_cores + lax.axis_index("c")
        base = wid * b_per_w
        pltpu.sync_copy(idx_hbm.at[pl.ds(base, b_per_w)], idx_v)
        pltpu.async_copy(table_hbm.at[idx_v], rows_v, sem).wait()
        pltpu.sync_copy(rows_v, out_hbm.at[pl.ds(base, b_per_w)])

    return gather_kernel


def _phi_kernel(s_ref, w1_ref, b1_ref, w2_ref, b2_ref, phi_ref):
    s = s_ref[...]
    h = jnp.dot(s, w1_ref[...], preferred_element_type=jnp.float32) + b1_ref[...]
    h = h * jax.nn.sigmoid(h)
    phi_ref[...] = (jnp.dot(h, w2_ref[...], preferred_element_type=jnp.float32)
                    + b2_ref[...]).astype(jnp.bfloat16)


def _pair_kernel(meta_ref, pos_row_ref, batch_row_ref, posc_ref, batchc_ref,
                 phi1_ref, phi3_ref, w1_ref, w3_ref, b1_ref, b3_ref,
                 ds_ref, dv_ref, ls_a, lv_a, rs_a, rv_a):
    r = pl.program_id(0)
    start_col = meta_ref[0, r]
    num_chunks = meta_ref[1, r]

    pos_row = pos_row_ref[...]
    batch_row = batch_row_ref[...]
    sq_i = jnp.sum(pos_row * pos_row, axis=1, keepdims=True)
    row_ids = r * _RB + jax.lax.broadcasted_iota(jnp.int32, (_RB, _CW), 0)

    b1 = b1_ref[...]
    b3 = b3_ref[...]
    bf = jnp.bfloat16

    def build(w, ls_ref, lv_ref, rs_ref, rv_ref):
        w = pl.multiple_of(w, 8)
        posc = posc_ref[pl.ds(w, _CW), :].T
        batchc = batchc_ref[pl.ds(w, _CW), :].reshape(1, _CW)
        phi1 = phi1_ref[pl.ds(w, _CW), :]
        phi3 = phi3_ref[pl.ds(w, _CW), :]

        sq_j = jnp.sum(posc * posc, axis=0, keepdims=True)
        cross = jnp.dot(pos_row, posc, preferred_element_type=jnp.float32)
        d2m = jnp.maximum(sq_i + sq_j - 2.0 * cross, 0.0)
        dm = jnp.sqrt(d2m)

        col_ids = w + jax.lax.broadcasted_iota(jnp.int32, (_RB, _CW), 1)
        m = ((dm <= _CUTOFF)
             & (batch_row == batchc)
             & (row_ids != col_ids))
        mf = m.astype(jnp.float32)

        rel0 = pos_row[:, 0:1] - posc[0:1, :]
        rel1 = pos_row[:, 1:2] - posc[1:2, :]
        rel2 = pos_row[:, 2:3] - posc[2:3, :]
        d2g = rel0 * rel0 + rel1 * rel1 + rel2 * rel2
        dg = jnp.sqrt(d2g)

        d_safe = jnp.where(m, dg, 1.0)
        invd_b = (mf / d_safe).astype(bf)
        relm0_b = (rel0 * mf).astype(bf)
        relm1_b = (rel1 * mf).astype(bf)
        relm2_b = (rel2 * mf).astype(bf)
        dg_b = dg.astype(bf)

        x = dg * (np.pi / _CUTOFF)
        s_cur = jnp.sin(x)
        cos2 = 2.0 * jnp.cos(x)
        s_prev = jnp.zeros_like(s_cur)

        for k in range(_NFREQ):
            ks = slice(k * _CW, (k + 1) * _CW)
            s_b = s_cur.astype(bf)
            ls_ref[:, ks] = s_b * invd_b
            lv_ref[0 * _RB:1 * _RB, ks] = s_b * relm0_b
            lv_ref[1 * _RB:2 * _RB, ks] = s_b * relm1_b
            lv_ref[2 * _RB:3 * _RB, ks] = s_b * relm2_b
            rs_ref[ks, :] = phi1 * w1_ref[k:k + 1, :]
            rv_ref[ks, :] = phi3 * w3_ref[k:k + 1, :]
            s_prev, s_cur = s_cur, cos2 * s_cur - s_prev
        kb = slice(_NFREQ * _CW, (_NFREQ + 1) * _CW)
        ls_ref[:, kb] = mf.astype(bf)
        lv_ref[0 * _RB:1 * _RB, kb] = dg_b * relm0_b
        lv_ref[1 * _RB:2 * _RB, kb] = dg_b * relm1_b
        lv_ref[2 * _RB:3 * _RB, kb] = dg_b * relm2_b
        rs_ref[kb, :] = phi1 * b1
        rv_ref[kb, :] = phi3 * b3

    def chunk_body(t, carry):
        acc_s, acc_v = carry
        build(start_col + t * _CW, ls_a, lv_a, rs_a, rv_a)
        acc_s = acc_s + jnp.dot(ls_a[...], rs_a[...],
                                preferred_element_type=jnp.float32)
        acc_v = acc_v + jnp.dot(lv_a[...], rv_a[...],
                                preferred_element_type=jnp.float32)
        return acc_s, acc_v

    acc_s, acc_v = jax.lax.fori_loop(
        0, num_chunks, chunk_body,
        (jnp.zeros((_RB, _D), jnp.float32),
         jnp.zeros((3 * _RB, _D), jnp.float32)))

    ds_ref[...] = acc_s
    dv_ref[0] = acc_v[0 * _RB:1 * _RB, :]
    dv_ref[1] = acc_v[1 * _RB:2 * _RB, :]
    dv_ref[2] = acc_v[2 * _RB:3 * _RB, :]


@functools.partial(jax.jit, static_argnums=())
def kernel(z, pos, batch, emb_table, W_phi1, b_phi1, W_phi2, b_phi2, W_rbf, b_rbf):
    z = z.astype(jnp.int32)
    batch = batch.astype(jnp.int32)

    s = _make_emb_gather()(emb_table, z)
    w2r = jnp.concatenate([W_phi2[:, :_D], W_phi2[:, 2 * _D:]], axis=1)
    b2r = jnp.concatenate([b_phi2[:_D], b_phi2[2 * _D:]]).reshape(1, 2 * _D)
    phi = pl.pallas_call(
        _phi_kernel,
        grid=(_N // 256,),
        in_specs=[
            pl.BlockSpec((256, _D), lambda i: (i, 0)),
            pl.BlockSpec((_D, _D), lambda i: (0, 0)),
            pl.BlockSpec((1, _D), lambda i: (0, 0)),
            pl.BlockSpec((_D, 2 * _D), lambda i: (0, 0)),
            pl.BlockSpec((1, 2 * _D), lambda i: (0, 0)),
        ],
        out_specs=pl.BlockSpec((256, 2 * _D), lambda i: (i, 0)),
        out_shape=jax.ShapeDtypeStruct((_N, 2 * _D), jnp.bfloat16),
    )(s, W_phi1, b_phi1.reshape(1, _D), w2r, b2r)

    phi1 = jnp.zeros((_NP, _D), jnp.bfloat16).at[:_N].set(phi[:, :_D])
    phi3 = jnp.zeros((_NP, _D), jnp.bfloat16).at[:_N].set(phi[:, _D:])
    posc = jnp.zeros((_NP, 3), jnp.float32).at[:_N].set(pos)
    batchc = jnp.full((_NP, 1), -1, jnp.int32).at[:_N, 0].set(batch)

    b_first = batch[::_RB]
    b_last = batch[_RB - 1::_RB]
    c_lo = jnp.searchsorted(batch, b_first, side="left").astype(jnp.int32)
    c_hi = jnp.searchsorted(batch, b_last, side="right").astype(jnp.int32)
    start_col = (c_lo // 8) * 8
    num_chunks = (c_hi - start_col + _CW - 1) // _CW
    meta = jnp.stack([start_col, num_chunks]).astype(jnp.int32)

    w1 = W_rbf[:, :_D].astype(jnp.bfloat16)
    w3 = W_rbf[:, 2 * _D:].astype(jnp.bfloat16)
    b1 = b_rbf[:_D].reshape(1, _D).astype(jnp.bfloat16)
    b3 = b_rbf[2 * _D:].reshape(1, _D).astype(jnp.bfloat16)

    grid_spec = pltpu.PrefetchScalarGridSpec(
        num_scalar_prefetch=1,
        grid=(_NRB,),
        in_specs=[
            pl.BlockSpec((_RB, 3), lambda r, *_: (r, 0)),
            pl.BlockSpec((_RB, 1), lambda r, *_: (r, 0)),
            pl.BlockSpec((_NP, 3), lambda r, *_: (0, 0)),
            pl.BlockSpec((_NP, 1), lambda r, *_: (0, 0)),
            pl.BlockSpec((_NP, _D), lambda r, *_: (0, 0)),
            pl.BlockSpec((_NP, _D), lambda r, *_: (0, 0)),
            pl.BlockSpec((_NFREQ, _D), lambda r, *_: (0, 0)),
            pl.BlockSpec((_NFREQ, _D), lambda r, *_: (0, 0)),
            pl.BlockSpec((1, _D), lambda r, *_: (0, 0)),
            pl.BlockSpec((1, _D), lambda r, *_: (0, 0)),
        ],
        out_specs=[
            pl.BlockSpec((_RB, _D), lambda r, *_: (r, 0)),
            pl.BlockSpec((3, _RB, _D), lambda r, *_: (0, r, 0)),
        ],
        scratch_shapes=[
            pltpu.VMEM((_RB, _KW), jnp.bfloat16),
            pltpu.VMEM((3 * _RB, _KW), jnp.bfloat16),
            pltpu.VMEM((_KW, _D), jnp.bfloat16),
            pltpu.VMEM((_KW, _D), jnp.bfloat16),
        ],
    )
    delta_s, delta_v = pl.pallas_call(
        _pair_kernel,
        grid_spec=grid_spec,
        out_shape=[
            jax.ShapeDtypeStruct((_N, _D), jnp.float32),
            jax.ShapeDtypeStruct((3, _N, _D), jnp.float32),
        ],
    )(meta, pos, batch.reshape(_N, 1), posc, batchc, phi1, phi3, w1, w3, b1, b3)

    return delta_s, delta_v
